# baseline (speedup 1.0000x reference)
import functools

import jax
import jax.numpy as jnp
from jax import lax
from jax.experimental import pallas as pl
from jax.experimental.pallas import tpu as pltpu
from jax.experimental.pallas import tpu_sc as plsc

NUM_USERS = 100000
EMBED_DIM = 64
BATCH = 16384

_NC = 2
_NS = 16
_NW = _NC * _NS
_B_PER_W = BATCH // _NW
_CHUNK = 128
_N_CHUNKS = _B_PER_W // _CHUNK

_mesh = plsc.VectorSubcoreMesh(core_axis_name="c", subcore_axis_name="s")


@functools.partial(
    pl.kernel,
    mesh=_mesh,
    out_type=jax.ShapeDtypeStruct((BATCH, EMBED_DIM), jnp.float32),
    scratch_types=[
        pltpu.VMEM((_B_PER_W,), jnp.int32),
        pltpu.VMEM((_B_PER_W, EMBED_DIM), jnp.float32),
        pltpu.SemaphoreType.DMA,
    ],
    compiler_params=pltpu.CompilerParams(use_tc_tiling_on_sc=False),
)
def _gather_kernel(idx_hbm, table_hbm, out_hbm, idx_v, rows_v, sem):
    wid = lax.axis_index("s") * _NC + lax.axis_index("c")
    base = wid * _B_PER_W
    pltpu.sync_copy(idx_hbm.at[pl.ds(base, _B_PER_W)], idx_v)
    copies = []
    for j in range(_N_CHUNKS):
        copies.append(
            pltpu.async_copy(
                table_hbm.at[idx_v.at[pl.ds(j * _CHUNK, _CHUNK)]],
                rows_v.at[pl.ds(j * _CHUNK, _CHUNK)],
                sem,
            )
        )
    for c in copies:
        c.wait()
    pltpu.sync_copy(rows_v, out_hbm.at[pl.ds(base, _B_PER_W)])


def kernel(idx, embedding_table):
    return _gather_kernel(idx.astype(jnp.int32), embedding_table)


# speedup vs baseline: 1.4883x; 1.4883x over previous
import functools

import jax
import jax.numpy as jnp
from jax import lax
from jax.experimental import pallas as pl
from jax.experimental.pallas import tpu as pltpu
from jax.experimental.pallas import tpu_sc as plsc

NUM_USERS = 100000
EMBED_DIM = 64
BATCH = 16384

_NC = 2
_NS = 16
_NW = _NC * _NS
_B_PER_W = BATCH // _NW

_mesh = plsc.VectorSubcoreMesh(core_axis_name="c", subcore_axis_name="s")


@functools.partial(
    pl.kernel,
    mesh=_mesh,
    out_type=jax.ShapeDtypeStruct((BATCH, EMBED_DIM), jnp.float32),
    scratch_types=[
        pltpu.VMEM((_B_PER_W,), jnp.int32),
        pltpu.VMEM((_B_PER_W, EMBED_DIM), jnp.float32),
        pltpu.SemaphoreType.DMA,
    ],
)
def _gather_kernel(idx_hbm, table_hbm, out_hbm, idx_v, rows_v, sem):
    wid = lax.axis_index("s") * _NC + lax.axis_index("c")
    base = wid * _B_PER_W
    pltpu.sync_copy(idx_hbm.at[pl.ds(base, _B_PER_W)], idx_v)

    def fire(g, carry):
        vec = idx_v[pl.ds(g * 16, 16)]
        for l in range(16):
            pltpu.async_copy(table_hbm.at[vec[l]], rows_v.at[g * 16 + l], sem)
        return carry

    lax.fori_loop(0, _B_PER_W // 16, fire, 0)

    drain = pltpu.make_async_copy(table_hbm.at[0], rows_v.at[0], sem)

    def wait(j, carry):
        drain.wait()
        return carry

    lax.fori_loop(0, _B_PER_W, wait, 0, unroll=8)
    pltpu.sync_copy(rows_v, out_hbm.at[pl.ds(base, _B_PER_W)])


def kernel(idx, embedding_table):
    return _gather_kernel(idx.astype(jnp.int32), embedding_table)


# speedup vs baseline: 1.6988x; 1.1414x over previous
import functools

import jax
import jax.numpy as jnp
from jax import lax
from jax.experimental import pallas as pl
from jax.experimental.pallas import tpu as pltpu
from jax.experimental.pallas import tpu_sc as plsc

NUM_USERS = 100000
EMBED_DIM = 64
BATCH = 16384

_NC = 2
_NS = 16
_NW = _NC * _NS
_D_PER_W = EMBED_DIM // _NW
_CHUNK = 4096
_N_CHUNKS = BATCH // _CHUNK

_mesh = plsc.VectorSubcoreMesh(core_axis_name="c", subcore_axis_name="s")


@functools.partial(
    pl.kernel,
    mesh=_mesh,
    out_type=jax.ShapeDtypeStruct((EMBED_DIM, BATCH), jnp.float32),
    scratch_types=[
        pltpu.VMEM((NUM_USERS,), jnp.float32),
        pltpu.VMEM((_CHUNK,), jnp.int32),
        pltpu.VMEM((_CHUNK,), jnp.float32),
        pltpu.SemaphoreType.DMA,
    ],
    compiler_params=pltpu.CompilerParams(needs_layout_passes=False),
)
def _gather_kernel(idx_hbm, tab_t_hbm, out_t_hbm, row_v, idx_v, out_v, sem):
    wid = lax.axis_index("s") * _NC + lax.axis_index("c")

    def per_dim(r, carry):
        d = wid * _D_PER_W + r
        pltpu.sync_copy(tab_t_hbm.at[d], row_v)

        def per_chunk(h, carry2):
            pltpu.sync_copy(idx_hbm.at[pl.ds(h * _CHUNK, _CHUNK)], idx_v)

            def gather16(k, carry3):
                vec = idx_v[pl.ds(k * 16, 16)]
                out_v[pl.ds(k * 16, 16)] = plsc.load_gather(row_v, [vec])
                return carry3

            lax.fori_loop(0, _CHUNK // 16, gather16, 0, unroll=8)
            pltpu.sync_copy(out_v, out_t_hbm.at[d, pl.ds(h * _CHUNK, _CHUNK)])
            return carry2

        lax.fori_loop(0, _N_CHUNKS, per_chunk, 0)
        return carry

    lax.fori_loop(0, _D_PER_W, per_dim, 0)


def kernel(idx, embedding_table):
    out_t = _gather_kernel(idx.astype(jnp.int32), embedding_table.T)
    return out_t.T


# speedup vs baseline: 2.6366x; 1.5520x over previous
import functools

import jax
import jax.numpy as jnp
from jax import lax
from jax.experimental import pallas as pl
from jax.experimental.pallas import tpu as pltpu
from jax.experimental.pallas import tpu_sc as plsc

NUM_USERS = 100000
EMBED_DIM = 64
BATCH = 16384

_NC = 2
_NS = 16
_NW = _NC * _NS
_D_PER_W = EMBED_DIM // _NW
_CHUNK = 4096
_N_CHUNKS = BATCH // _CHUNK
_GRP = 8

_mesh = plsc.VectorSubcoreMesh(core_axis_name="c", subcore_axis_name="s")


@functools.partial(
    pl.kernel,
    mesh=_mesh,
    out_type=jax.ShapeDtypeStruct((EMBED_DIM, BATCH), jnp.float32),
    scratch_types=[
        pltpu.VMEM((NUM_USERS,), jnp.float32),
        pltpu.VMEM((BATCH,), jnp.int32),
        pltpu.VMEM((2, _CHUNK), jnp.float32),
        pltpu.SemaphoreType.DMA,
        pltpu.SemaphoreType.DMA,
    ],
    compiler_params=pltpu.CompilerParams(needs_layout_passes=False),
)
def _gather_kernel(idx_hbm, tab_t_hbm, out_t_hbm, row_v, idx_v, out_v,
                   sem_idx, sem_out):
    wid = lax.axis_index("s") * _NC + lax.axis_index("c")

    idx_cp = pltpu.async_copy(idx_hbm, idx_v, sem_idx)

    out_cps = [None, None]
    for r in range(_D_PER_W):
        d = wid * _D_PER_W + r
        pltpu.sync_copy(tab_t_hbm.at[d], row_v)
        if r == 0:
            idx_cp.wait()
        for h in range(_N_CHUNKS):
            buf = h % 2
            if out_cps[buf] is not None:
                out_cps[buf].wait()
                out_cps[buf] = None

            def gather_block(k, carry, h=h, buf=buf):
                base = h * _CHUNK + k * (16 * _GRP)
                vecs = [idx_v[pl.ds(base + 16 * j, 16)] for j in range(_GRP)]
                gs = [plsc.load_gather(row_v, [v]) for v in vecs]
                off = k * (16 * _GRP)
                for j in range(_GRP):
                    out_v[buf, pl.ds(off + 16 * j, 16)] = gs[j]
                return carry

            lax.fori_loop(0, _CHUNK // (16 * _GRP), gather_block, 0)
            out_cps[buf] = pltpu.async_copy(
                out_v.at[buf],
                out_t_hbm.at[d, pl.ds(h * _CHUNK, _CHUNK)],
                sem_out,
            )
    for cp in out_cps:
        if cp is not None:
            cp.wait()


def kernel(idx, embedding_table):
    out_t = _gather_kernel(idx.astype(jnp.int32), embedding_table.T)
    return out_t.T


# speedup vs baseline: 2.6476x; 1.0042x over previous
import functools

import jax
import jax.numpy as jnp
from jax import lax
from jax.experimental import pallas as pl
from jax.experimental.pallas import tpu as pltpu
from jax.experimental.pallas import tpu_sc as plsc

NUM_USERS = 100000
EMBED_DIM = 64
BATCH = 16384

_NC = 2
_NS = 16
_NW = _NC * _NS
_D_PER_W = EMBED_DIM // _NW
_CHUNK = 4096
_N_CHUNKS = BATCH // _CHUNK
_GRP = 16

_mesh = plsc.VectorSubcoreMesh(core_axis_name="c", subcore_axis_name="s")


@functools.partial(
    pl.kernel,
    mesh=_mesh,
    out_type=jax.ShapeDtypeStruct((EMBED_DIM, BATCH), jnp.float32),
    scratch_types=[
        pltpu.VMEM((NUM_USERS,), jnp.float32),
        pltpu.VMEM((BATCH,), jnp.int32),
        pltpu.VMEM((2, _CHUNK), jnp.float32),
        pltpu.VMEM_SHARED((BATCH,), jnp.int32),
        pltpu.SemaphoreType.DMA,
        pltpu.SemaphoreType.DMA,
    ],
    compiler_params=pltpu.CompilerParams(needs_layout_passes=False),
)
def _gather_kernel(idx_hbm, tab_t_hbm, out_t_hbm, row_v, idx_v, out_v,
                   idx_sh, sem_idx, sem_out):
    wid = lax.axis_index("s") * _NC + lax.axis_index("c")
    sid = lax.axis_index("s")

    @pl.when(sid == 0)
    def _():
        pltpu.sync_copy(idx_hbm, idx_sh)

    out_cps = [None, None]
    for r in range(_D_PER_W):
        d = wid * _D_PER_W + r
        pltpu.sync_copy(tab_t_hbm.at[d], row_v)
        if r == 0:
            plsc.subcore_barrier()
            pltpu.async_copy(idx_sh, idx_v, sem_idx).wait()
        for h in range(_N_CHUNKS):
            buf = h % 2
            if out_cps[buf] is not None:
                out_cps[buf].wait()
                out_cps[buf] = None

            def gather_block(k, carry, h=h, buf=buf):
                base = h * _CHUNK + k * (16 * _GRP)
                vecs = [idx_v[pl.ds(base + 16 * j, 16)] for j in range(_GRP)]
                gs = [plsc.load_gather(row_v, [v]) for v in vecs]
                off = k * (16 * _GRP)
                for j in range(_GRP):
                    out_v[buf, pl.ds(off + 16 * j, 16)] = gs[j]
                return carry

            lax.fori_loop(0, _CHUNK // (16 * _GRP), gather_block, 0)
            out_cps[buf] = pltpu.async_copy(
                out_v.at[buf],
                out_t_hbm.at[d, pl.ds(h * _CHUNK, _CHUNK)],
                sem_out,
            )
    for cp in out_cps:
        if cp is not None:
            cp.wait()


def kernel(idx, embedding_table):
    out_t = _gather_kernel(idx.astype(jnp.int32), embedding_table.T)
    return out_t.T
